# baseline (speedup 1.0000x reference)
import functools

import jax
import jax.numpy as jnp
from jax import lax
from jax.experimental import pallas as pl
from jax.experimental.pallas import tpu as pltpu
from jax.experimental.pallas import tpu_sc as plsc

B = 16384
NC, NS = 2, 16
NW = NC * NS
BPW = B // NW
CHUNK = 128
NCHUNK = BPW // CHUNK

_MESH = plsc.VectorSubcoreMesh(core_axis_name="c", subcore_axis_name="s",
                               num_cores=NC, num_subcores=NS)


def _sc_gather_body(emb_u, emb_t, idx_u, idx_t, out_u, out_t,
                    vi_u, vi_t, r_u, r_t, sem):
    wid = lax.axis_index("s") * NC + lax.axis_index("c")
    base = wid * BPW

    pltpu.sync_copy(idx_u.at[pl.ds(base, BPW)], vi_u)
    pltpu.sync_copy(idx_t.at[pl.ds(base, BPW)], vi_t)

    copies = []
    for iv, tbl, rv in ((vi_u, emb_u, r_u), (vi_t, emb_t, r_t)):
        for c in range(NCHUNK):
            copies.append(
                pltpu.async_copy(tbl.at[iv.at[pl.ds(c * CHUNK, CHUNK)]],
                                 rv.at[pl.ds(c * CHUNK, CHUNK), :], sem))
    for cp in copies:
        cp.wait()

    pltpu.sync_copy(r_u, out_u.at[pl.ds(base, BPW), :])
    pltpu.sync_copy(r_t, out_t.at[pl.ds(base, BPW), :])


_sc_gather = functools.partial(
    pl.kernel,
    out_type=(jax.ShapeDtypeStruct((B, 32), jnp.float32),
              jax.ShapeDtypeStruct((B, 16), jnp.float32)),
    mesh=_MESH,
    scratch_types=(
        pltpu.VMEM((BPW,), jnp.int32),
        pltpu.VMEM((BPW,), jnp.int32),
        pltpu.VMEM((BPW, 32), jnp.float32),
        pltpu.VMEM((BPW, 16), jnp.float32),
        pltpu.SemaphoreType.DMA,
    ),
    compiler_params=pltpu.CompilerParams(use_tc_tiling_on_sc=False),
)(_sc_gather_body)


BLK = 512


def _onehot(idx, n):
    lanes = lax.broadcasted_iota(jnp.int32, (1, n), 1)
    return jnp.where(idx == lanes, 1.0, 0.0).astype(jnp.float32)


def _mlp_body(u, t, cl, gr, ar, rg, ten,
              w1u, w1t, w1c, w1g, w1a, w1r, w1ten, b1,
              ec, eg, ea, er, w2, b2, w3, b3, wp, bp, out):
    f32 = jnp.float32
    dot = functools.partial(jnp.dot, preferred_element_type=f32)

    h = ten[...] * w1ten[...] + b1[...]
    h = h + dot(u[...], w1u[...])
    h = h + dot(t[...], w1t[...])
    h = h + dot(dot(_onehot(cl[...], 128), ec[...]), w1c[...])
    h = h + dot(dot(_onehot(gr[...], 1024), eg[...]), w1g[...])
    h = h + dot(dot(_onehot(ar[...], 128), ea[...]), w1a[...])
    h = h + dot(dot(_onehot(rg[...], 128), er[...]), w1r[...])
    h = jnp.maximum(h, 0.0)
    h = jnp.maximum(dot(h, w2[...]) + b2[...], 0.0)
    h = jnp.maximum(dot(h, w3[...]) + b3[...], 0.0)
    out[...] = dot(h, wp[...]) + bp[...]


def _mlp(args):
    full = lambda shape: pl.BlockSpec(shape, lambda i: (0, 0))
    return pl.pallas_call(
        _mlp_body,
        grid=(B // BLK,),
        in_specs=(
            [pl.BlockSpec((BLK, 32), lambda i: (i, 0)),
             pl.BlockSpec((BLK, 16), lambda i: (i, 0))]
            + [pl.BlockSpec((BLK, 1), lambda i: (i, 0))] * 5
            + [full((32, 256)), full((16, 256)), full((8, 256)),
               full((8, 256)), full((4, 256)), full((4, 256)),
               full((1, 256)), full((1, 256)),
               full((128, 8)), full((1024, 8)), full((128, 4)),
               full((128, 4)),
               full((256, 128)), full((1, 128)), full((128, 64)),
               full((1, 64)), full((64, 64)), full((1, 64))]
        ),
        out_specs=pl.BlockSpec((BLK, 64), lambda i: (i, 0)),
        out_shape=jax.ShapeDtypeStruct((B, 64), jnp.float32),
    )(*args)


def kernel(CustomerCode, TownName, Cluster, GroupHeaderName, Area,
           RegionCategory, TenureYears,
           emb_user, emb_town, emb_cluster, emb_group, emb_area, emb_region,
           W1, b1, W2, b2, W3, b3, Wp, bp):
    zero = b1[0] * 0.0
    u, t = _sc_gather(emb_user + zero, emb_town + zero,
                      CustomerCode, TownName)

    col = lambda ix: ix.reshape(B, 1)
    padv = lambda tb, v: jnp.pad(tb, ((0, v - tb.shape[0]), (0, 0)))
    args = (
        u, t,
        col(Cluster), col(GroupHeaderName), col(Area), col(RegionCategory),
        TenureYears.reshape(B, 1),
        W1[0:32], W1[32:48], W1[48:56], W1[56:64], W1[64:68], W1[68:72],
        W1[72:73], b1.reshape(1, 256),
        padv(emb_cluster, 128), padv(emb_group, 1024), padv(emb_area, 128),
        padv(emb_region, 128),
        W2, b2.reshape(1, 128), W3, b3.reshape(1, 64),
        Wp, bp.reshape(1, 64),
    )
    return _mlp(args)


# speedup vs baseline: 1.5680x; 1.5680x over previous
import functools

import jax
import jax.numpy as jnp
from jax import lax
from jax.experimental import pallas as pl
from jax.experimental.pallas import tpu as pltpu
from jax.experimental.pallas import tpu_sc as plsc

B = 16384
NC, NS = 2, 16
NW = NC * NS
BPW = B // NW
CHUNK = 128
NCHUNK = BPW // CHUNK

_MESH = plsc.VectorSubcoreMesh(core_axis_name="c", subcore_axis_name="s",
                               num_cores=NC, num_subcores=NS)


def _sc_gather_body(emb_u, emb_t, idx_u, idx_t, out_u, out_t,
                    vi_u, vi_t, rows, sem):
    wid = lax.axis_index("s") * NC + lax.axis_index("c")
    base = wid * BPW

    pltpu.sync_copy(idx_u.at[pl.ds(base, BPW)], vi_u)
    pltpu.sync_copy(idx_t.at[pl.ds(base, BPW)], vi_t)

    for iv, tbl, out in ((vi_u, emb_u, out_u), (vi_t, emb_t, out_t)):
        copies = [
            pltpu.async_copy(tbl.at[iv.at[pl.ds(c * CHUNK, CHUNK)]],
                             rows.at[pl.ds(c * CHUNK, CHUNK), :], sem)
            for c in range(NCHUNK)
        ]
        for cp in copies:
            cp.wait()
        pltpu.sync_copy(rows, out.at[pl.ds(base, BPW), :])


_sc_gather = functools.partial(
    pl.kernel,
    out_type=(jax.ShapeDtypeStruct((B, 128), jnp.float32),
              jax.ShapeDtypeStruct((B, 128), jnp.float32)),
    mesh=_MESH,
    scratch_types=(
        pltpu.VMEM((BPW,), jnp.int32),
        pltpu.VMEM((BPW,), jnp.int32),
        pltpu.VMEM((BPW, 128), jnp.float32),
        pltpu.SemaphoreType.DMA,
    ),
)(_sc_gather_body)


BLK = 512


def _onehot(idx, n):
    lanes = lax.broadcasted_iota(jnp.int32, (1, n), 1)
    return jnp.where(idx == lanes, 1.0, 0.0).astype(jnp.float32)


def _mlp_body(u, t, cl, gr, ar, rg, ten,
              w1u, w1t, w1c, w1g, w1a, w1r, w1ten, b1,
              ec, eg, ea, er, w2, b2, w3, b3, wp, bp, out):
    f32 = jnp.float32
    dot = functools.partial(jnp.dot, preferred_element_type=f32)

    h = ten[...] * w1ten[...] + b1[...]
    h = h + dot(u[...], w1u[...])
    h = h + dot(t[...], w1t[...])
    h = h + dot(dot(_onehot(cl[...], 128), ec[...]), w1c[...])
    h = h + dot(dot(_onehot(gr[...], 1024), eg[...]), w1g[...])
    h = h + dot(dot(_onehot(ar[...], 128), ea[...]), w1a[...])
    h = h + dot(dot(_onehot(rg[...], 128), er[...]), w1r[...])
    h = jnp.maximum(h, 0.0)
    h = jnp.maximum(dot(h, w2[...]) + b2[...], 0.0)
    h = jnp.maximum(dot(h, w3[...]) + b3[...], 0.0)
    out[...] = dot(h, wp[...]) + bp[...]


def _mlp(args):
    blk = lambda w: pl.BlockSpec((BLK, w), lambda i: (i, 0))
    full = lambda shape: pl.BlockSpec(shape, lambda i: (0, 0))
    return pl.pallas_call(
        _mlp_body,
        grid=(B // BLK,),
        in_specs=(
            [blk(128), blk(128)]
            + [blk(1)] * 5
            + [full((128, 256)), full((128, 256)), full((8, 256)),
               full((8, 256)), full((4, 256)), full((4, 256)),
               full((1, 256)), full((1, 256)),
               full((128, 8)), full((1024, 8)), full((128, 4)),
               full((128, 4)),
               full((256, 128)), full((1, 128)), full((128, 64)),
               full((1, 64)), full((64, 64)), full((1, 64))]
        ),
        out_specs=pl.BlockSpec((BLK, 64), lambda i: (i, 0)),
        out_shape=jax.ShapeDtypeStruct((B, 64), jnp.float32),
    )(*args)


def kernel(CustomerCode, TownName, Cluster, GroupHeaderName, Area,
           RegionCategory, TenureYears,
           emb_user, emb_town, emb_cluster, emb_group, emb_area, emb_region,
           W1, b1, W2, b2, W3, b3, Wp, bp):
    u128 = jnp.pad(emb_user.T, ((0, 96), (0, 0))).T
    t128 = jnp.pad(emb_town.T, ((0, 112), (0, 0))).T
    u, t = _sc_gather(u128, t128, CustomerCode, TownName)

    col = lambda ix: ix.reshape(B, 1)
    padv = lambda tb, v: jnp.pad(tb, ((0, v - tb.shape[0]), (0, 0)))
    padw = lambda w: jnp.pad(w, ((0, 128 - w.shape[0]), (0, 0)))
    args = (
        u, t,
        col(Cluster), col(GroupHeaderName), col(Area), col(RegionCategory),
        TenureYears.reshape(B, 1),
        padw(W1[0:32]), padw(W1[32:48]), W1[48:56], W1[56:64], W1[64:68],
        W1[68:72], W1[72:73], b1.reshape(1, 256),
        padv(emb_cluster, 128), padv(emb_group, 1024), padv(emb_area, 128),
        padv(emb_region, 128),
        W2, b2.reshape(1, 128), W3, b3.reshape(1, 64),
        Wp, bp.reshape(1, 64),
    )
    return _mlp(args)


# speedup vs baseline: 1.6086x; 1.0259x over previous
import functools

import jax
import jax.numpy as jnp
from jax import lax
from jax.experimental import pallas as pl
from jax.experimental.pallas import tpu as pltpu
from jax.experimental.pallas import tpu_sc as plsc

B = 16384
NC, NS = 2, 16
NW = NC * NS
BPW = B // NW
CHUNK = 128
NCHUNK = BPW // CHUNK

_MESH = plsc.VectorSubcoreMesh(core_axis_name="c", subcore_axis_name="s",
                               num_cores=NC, num_subcores=NS)


def _sc_gather_body(emb_u, emb_t, idx_u, idx_t, out_u, out_t,
                    vi_u, vi_t, rows, sem):
    wid = lax.axis_index("s") * NC + lax.axis_index("c")
    base = wid * BPW

    pltpu.sync_copy(idx_u.at[pl.ds(base, BPW)], vi_u)
    pltpu.sync_copy(idx_t.at[pl.ds(base, BPW)], vi_t)

    for iv, tbl, out in ((vi_u, emb_u, out_u), (vi_t, emb_t, out_t)):
        copies = [
            pltpu.async_copy(tbl.at[iv.at[pl.ds(c * CHUNK, CHUNK)]],
                             rows.at[pl.ds(c * CHUNK, CHUNK), :], sem)
            for c in range(NCHUNK)
        ]
        for cp in copies:
            cp.wait()
        pltpu.sync_copy(rows, out.at[pl.ds(base, BPW), :])


_sc_gather = functools.partial(
    pl.kernel,
    out_type=(jax.ShapeDtypeStruct((B, 128), jnp.float32),
              jax.ShapeDtypeStruct((B, 128), jnp.float32)),
    mesh=_MESH,
    scratch_types=(
        pltpu.VMEM((BPW,), jnp.int32),
        pltpu.VMEM((BPW,), jnp.int32),
        pltpu.VMEM((BPW, 128), jnp.float32),
        pltpu.SemaphoreType.DMA,
    ),
)(_sc_gather_body)


BLK = 512


def _onehot(idx, n):
    lanes = lax.broadcasted_iota(jnp.int32, (1, n), 1)
    return jnp.where(idx == lanes, 1.0, 0.0).astype(jnp.float32)


def _mlp_body(u, t, ix4, ten,
              w1u, w1t, w1c, w1g, w1a, w1r, w1ten, b1,
              ec, eg, ea, er, w2, b2, w3, b3, wp, bp, out):
    f32, bf = jnp.float32, jnp.bfloat16
    dot = lambda a, b: jnp.dot(a.astype(bf), b.astype(bf),
                               preferred_element_type=f32)

    cl, gr, ar, rg = (ix4[:, 0:1], ix4[:, 1:2], ix4[:, 2:3], ix4[:, 3:4])
    h = ten[...] * w1ten[...] + b1[...]
    h = h + dot(u[...], w1u[...])
    h = h + dot(t[...], w1t[...])
    h = h + dot(dot(_onehot(cl, 128), ec[...]), w1c[...])
    h = h + dot(dot(_onehot(gr, 1024), eg[...]), w1g[...])
    h = h + dot(dot(_onehot(ar, 128), ea[...]), w1a[...])
    h = h + dot(dot(_onehot(rg, 128), er[...]), w1r[...])
    h = jnp.maximum(h, 0.0)
    h = jnp.maximum(dot(h, w2[...]) + b2[...], 0.0)
    h = jnp.maximum(dot(h, w3[...]) + b3[...], 0.0)
    out[...] = dot(h, wp[...]) + bp[...]


def _mlp(args):
    blk = lambda w: pl.BlockSpec((BLK, w), lambda i: (i, 0))
    full = lambda shape: pl.BlockSpec(shape, lambda i: (0, 0))
    return pl.pallas_call(
        _mlp_body,
        grid=(B // BLK,),
        in_specs=(
            [blk(128), blk(128), blk(4), blk(1)]
            + [full((128, 256)), full((128, 256)), full((8, 256)),
               full((8, 256)), full((4, 256)), full((4, 256)),
               full((1, 256)), full((1, 256)),
               full((128, 8)), full((1024, 8)), full((128, 4)),
               full((128, 4)),
               full((256, 128)), full((1, 128)), full((128, 64)),
               full((1, 64)), full((64, 64)), full((1, 64))]
        ),
        out_specs=pl.BlockSpec((BLK, 64), lambda i: (i, 0)),
        out_shape=jax.ShapeDtypeStruct((B, 64), jnp.float32),
    )(*args)


def kernel(CustomerCode, TownName, Cluster, GroupHeaderName, Area,
           RegionCategory, TenureYears,
           emb_user, emb_town, emb_cluster, emb_group, emb_area, emb_region,
           W1, b1, W2, b2, W3, b3, Wp, bp):
    u128 = jnp.pad(emb_user.T, ((0, 96), (0, 0))).T
    t128 = jnp.pad(emb_town.T, ((0, 112), (0, 0))).T
    u, t = _sc_gather(u128, t128, CustomerCode, TownName)

    padv = lambda tb, v: jnp.pad(tb, ((0, v - tb.shape[0]), (0, 0)))
    padw = lambda w: jnp.pad(w, ((0, 128 - w.shape[0]), (0, 0)))
    ix4 = jnp.stack([Cluster, GroupHeaderName, Area, RegionCategory], axis=1)
    args = (
        u, t, ix4,
        TenureYears.reshape(B, 1),
        padw(W1[0:32]), padw(W1[32:48]), W1[48:56], W1[56:64], W1[64:68],
        W1[68:72], W1[72:73], b1.reshape(1, 256),
        padv(emb_cluster, 128), padv(emb_group, 1024), padv(emb_area, 128),
        padv(emb_region, 128),
        W2, b2.reshape(1, 128), W3, b3.reshape(1, 64),
        Wp, bp.reshape(1, 64),
    )
    return _mlp(args)
